# baseline (speedup 1.0000x reference)
import functools

import jax
import jax.numpy as jnp
from jax import lax
from jax.experimental import pallas as pl
from jax.experimental.pallas import tpu as pltpu
from jax.experimental.pallas import tpu_sc as plsc

DIM = 128
MAX_DIS = 1000
B = 100000
C = 128
NCHUNKS = -(-B // C)
LAST_BASE = B - C
NW = 32
ITERS = -(-NCHUNKS // NW)

_mesh = plsc.VectorSubcoreMesh(core_axis_name="c", subcore_axis_name="s")


@functools.partial(
    pl.kernel,
    mesh=_mesh,
    out_type=jax.ShapeDtypeStruct((B, DIM), jnp.float32),
    scratch_types=[
        pltpu.VMEM((C,), jnp.int32),
        pltpu.VMEM((C, DIM), jnp.float32),
        pltpu.SemaphoreType.DMA,
    ],
)
def _gather_kernel(d_hbm, embed_hbm, out_hbm, idx_v, rows_v, sem):
    wid = lax.axis_index("s") * 2 + lax.axis_index("c")

    def chunk_body(i, carry):
        g = wid + NW * i

        @pl.when(g < NCHUNKS)
        def _():
            base = jnp.minimum(g * C, LAST_BASE)
            pltpu.sync_copy(d_hbm.at[pl.ds(base, C)], idx_v)
            for j in range(C // 16):
                sl = pl.ds(j * 16, 16)
                idx_v[sl] = jnp.minimum(jnp.maximum(idx_v[sl], 0), MAX_DIS)
            pltpu.async_copy(embed_hbm.at[idx_v], rows_v, sem).wait()
            pltpu.sync_copy(rows_v, out_hbm.at[pl.ds(base, C)])

        return carry

    lax.fori_loop(0, ITERS, chunk_body, None)


def kernel(d, embed_d):
    return _gather_kernel(d, embed_d)
